# baseline (speedup 1.0000x reference)
import functools

import jax
import jax.numpy as jnp
from jax import lax
from jax.experimental import pallas as pl
from jax.experimental.pallas import tpu as pltpu
from jax.experimental.pallas import tpu_sc as plsc

N = 10000
E = 320000
D = 128
H = 128
ED = 16
G = 64

NC = 2
NS = 16
NW = NC * NS
EPW = E // NW
C = 40
NCHUNK = EPW // C
RPS = 624
RTAIL = N - NS * RPS


def _make_edge_gather_scatter(layer: int):
    mesh = plsc.VectorSubcoreMesh(core_axis_name="c", subcore_axis_name="s")

    @functools.partial(
        pl.kernel,
        out_type=jax.ShapeDtypeStruct((NC, N, D), jnp.float32),
        mesh=mesh,
        scratch_types=[
            [pltpu.VMEM((C,), jnp.int32) for _ in range(3)],
            [pltpu.VMEM((C,), jnp.int32) for _ in range(3)],
            [pltpu.VMEM((C, D), jnp.float32) for _ in range(3)],
            [pltpu.VMEM((C, D), jnp.float32) for _ in range(3)],
            pltpu.VMEM_SHARED((N, D), jnp.float32),
            pltpu.SemaphoreType.DMA((5, 3)),
        ],
    )
    def k(h_hbm, e3_hbm, src_hbm, dst_hbm, z_hbm, out_hbm,
          src_v, dst_v, hrow_v, e_v, aggr_sh, sems):
        c = lax.axis_index("c")
        s = lax.axis_index("s")
        wid = s * NC + c
        base = wid * EPW
        pltpu.sync_copy(z_hbm.at[pl.ds(s * RPS, RPS)],
                        aggr_sh.at[pl.ds(s * RPS, RPS)])

        @pl.when(s == NS - 1)
        def _():
            pltpu.sync_copy(z_hbm.at[pl.ds(NS * RPS, RTAIL)],
                            aggr_sh.at[pl.ds(NS * RPS, RTAIL)])

        plsc.subcore_barrier()

        def src_desc(j, b):
            return pltpu.make_async_copy(
                src_hbm.at[pl.ds(base + j * C, C)], src_v[b], sems.at[3, b])

        def gather_desc(b):
            return pltpu.make_async_copy(
                h_hbm.at[src_v[b]], hrow_v[b], sems.at[0, b])

        def e_desc(j, b):
            return pltpu.make_async_copy(
                e3_hbm.at[layer, pl.ds(base + j * C, C)], e_v[b],
                sems.at[1, b])

        def dst_desc(j, b):
            return pltpu.make_async_copy(
                dst_hbm.at[pl.ds(base + j * C, C)], dst_v[b],
                sems.at[2, b])

        def scat_wait(b):
            pltpu.make_async_copy(
                hrow_v[b], aggr_sh.at[dst_v[b]], sems.at[3, b]).wait()

        def issue(j, b):
            src_desc(j, b).start()
            dst_desc(j, b).start()
            e_desc(j, b).start()
            src_desc(j, b).wait()
            gather_desc(b).start()

        def consume(j, b):
            gather_desc(b).wait()
            e_desc(j, b).wait()

            @plsc.parallel_loop(0, C, unroll=2)
            def _(i):
                for col in range(D // 16):
                    sl = pl.ds(col * 16, 16)
                    hrow_v[b][i, sl] = jnp.maximum(
                        hrow_v[b][i, sl] + e_v[b][i, sl], 0.0)

            dst_desc(j, b).wait()
            pltpu.async_copy(hrow_v[b], aggr_sh.at[dst_v[b]],
                             sems.at[3, b], add=True)

        issue(0, 0)
        issue(1, 1)

        def trip(i, carry):
            j0 = 3 * i
            for t in range(3):
                j = j0 + t
                consume(j, t)
                b2 = (t + 2) % 3

                @pl.when((j >= 1) & (j + 2 < NCHUNK))
                def _():
                    scat_wait(b2)

                @pl.when(j + 2 < NCHUNK)
                def _():
                    issue(j + 2, b2)

            return carry

        lax.fori_loop(0, NCHUNK // 3, trip, 0)
        consume(NCHUNK - 1, (NCHUNK - 1) % 3)
        scat_wait((NCHUNK - 3) % 3)
        scat_wait((NCHUNK - 2) % 3)
        scat_wait((NCHUNK - 1) % 3)
        plsc.subcore_barrier()
        pltpu.sync_copy(aggr_sh.at[pl.ds(s * RPS, RPS)],
                        out_hbm.at[c, pl.ds(s * RPS, RPS)])

        @pl.when(s == NS - 1)
        def _():
            pltpu.sync_copy(aggr_sh.at[pl.ds(NS * RPS, RTAIL)],
                            out_hbm.at[c, pl.ds(NS * RPS, RTAIL)])

    return k


_EDGE_GS = [_make_edge_gather_scatter(l) for l in range(3)]


BE = 4000


def _edge_mlp_body(attr_ref, we_ref, be_ref, out_ref):
    a = attr_ref[...]
    for l in range(3):
        out_ref[l] = (jnp.dot(a, we_ref[l], preferred_element_type=jnp.float32)
                      + be_ref[l][None, :])


def _edge_mlp(edge_attr, we_s, be_s):
    return pl.pallas_call(
        _edge_mlp_body,
        grid=(E // BE,),
        in_specs=[
            pl.BlockSpec((BE, ED), lambda i: (i, 0)),
            pl.BlockSpec((3, ED, D), lambda i: (0, 0, 0)),
            pl.BlockSpec((3, D), lambda i: (0, 0)),
        ],
        out_specs=pl.BlockSpec((3, BE, D), lambda i: (0, i, 0)),
        out_shape=jax.ShapeDtypeStruct((3, E, D), jnp.float32),
    )(edge_attr, we_s, be_s)


BN = 1000


def _node_mlp_body(h_ref, p_ref, wa_ref, ba_ref, wb_ref, bb_ref,
                   y_ref, st_ref):
    i = pl.program_id(0)
    hpre = h_ref[...] + p_ref[0] + p_ref[1]
    t = jnp.maximum(
        jnp.dot(hpre, wa_ref[...], preferred_element_type=jnp.float32)
        + ba_ref[...], 0.0)
    y = jnp.dot(t, wb_ref[...], preferred_element_type=jnp.float32) + bb_ref[...]
    y_ref[...] = y
    s1 = jnp.sum(y, axis=0, keepdims=True)
    s2 = jnp.sum(y * y, axis=0, keepdims=True)
    upd = jnp.concatenate(
        [s1, s2, jnp.zeros((6, H), jnp.float32)], axis=0)

    @pl.when(i == 0)
    def _():
        st_ref[...] = upd

    @pl.when(i > 0)
    def _():
        st_ref[...] = st_ref[...] + upd


def _node_mlp(h, p, wa, ba, wb, bb):
    return pl.pallas_call(
        _node_mlp_body,
        grid=(N // BN,),
        in_specs=[
            pl.BlockSpec((BN, D), lambda i: (i, 0)),
            pl.BlockSpec((NC, BN, D), lambda i: (0, i, 0)),
            pl.BlockSpec((D, H), lambda i: (0, 0)),
            pl.BlockSpec((1, H), lambda i: (0, 0)),
            pl.BlockSpec((H, H), lambda i: (0, 0)),
            pl.BlockSpec((1, H), lambda i: (0, 0)),
        ],
        out_specs=[
            pl.BlockSpec((BN, H), lambda i: (i, 0)),
            pl.BlockSpec((8, H), lambda i: (0, 0)),
        ],
        out_shape=[
            jax.ShapeDtypeStruct((N, H), jnp.float32),
            jax.ShapeDtypeStruct((8, H), jnp.float32),
        ],
    )(h, p, wa, ba.reshape(1, H), wb, bb.reshape(1, H))


def _norm_body(y_ref, st_ref, g_ref, bt_ref, h_ref):
    mu = st_ref[0:1, :] * (1.0 / N)
    ex2 = st_ref[1:2, :] * (1.0 / N)
    var = ex2 - mu * mu
    scale = lax.rsqrt(var + 1e-5) * g_ref[...]
    h_ref[...] = jnp.maximum((y_ref[...] - mu) * scale + bt_ref[...], 0.0)


def _norm(y, st, g, bt):
    return pl.pallas_call(
        _norm_body,
        grid=(N // BN,),
        in_specs=[
            pl.BlockSpec((BN, H), lambda i: (i, 0)),
            pl.BlockSpec((8, H), lambda i: (0, 0)),
            pl.BlockSpec((1, H), lambda i: (0, 0)),
            pl.BlockSpec((1, H), lambda i: (0, 0)),
        ],
        out_specs=pl.BlockSpec((BN, H), lambda i: (i, 0)),
        out_shape=jax.ShapeDtypeStruct((N, H), jnp.float32),
    )(y, st, g.reshape(1, H), bt.reshape(1, H))


def _pool_body(h_ref, b_ref, wo_ref, bo_ref, out_ref, acc_ref, cnt_ref):
    i = pl.program_id(0)
    b = b_ref[0, 0, :]
    gids = lax.broadcasted_iota(jnp.int32, (G, BN), 0)
    onehot = (b[None, :] == gids).astype(jnp.float32)
    pacc = jnp.dot(onehot, h_ref[...], preferred_element_type=jnp.float32)
    pcnt = jnp.sum(onehot, axis=1, keepdims=True)

    @pl.when(i == 0)
    def _():
        acc_ref[...] = pacc
        cnt_ref[...] = pcnt

    @pl.when(i > 0)
    def _():
        acc_ref[...] = acc_ref[...] + pacc
        cnt_ref[...] = cnt_ref[...] + pcnt

    @pl.when(i == (N // BN) - 1)
    def _():
        pooled = acc_ref[...] / jnp.maximum(cnt_ref[...], 1.0)
        out_ref[...] = (jnp.dot(pooled, wo_ref[...],
                                preferred_element_type=jnp.float32)
                        + bo_ref[...])


def _pool(h, batch, wo, bo):
    b3 = batch.reshape(N // BN, 1, BN)
    return pl.pallas_call(
        _pool_body,
        grid=(N // BN,),
        in_specs=[
            pl.BlockSpec((BN, H), lambda i: (i, 0)),
            pl.BlockSpec((1, 1, BN), lambda i: (i, 0, 0)),
            pl.BlockSpec((H, 1), lambda i: (0, 0)),
            pl.BlockSpec((1, 1), lambda i: (0, 0)),
        ],
        out_specs=pl.BlockSpec((G, 1), lambda i: (0, 0)),
        out_shape=jax.ShapeDtypeStruct((G, 1), jnp.float32),
        scratch_shapes=[
            pltpu.VMEM((G, H), jnp.float32),
            pltpu.VMEM((G, 1), jnp.float32),
        ],
    )(h, b3, wo, bo.reshape(1, 1))


def kernel(x, edge_index, edge_attr, batch,
           We1, be1, W1a, b1a, W1b, b1b, gamma1, beta1,
           We2, be2, W2a, b2a, W2b, b2b, gamma2, beta2,
           We3, be3, W3a, b3a, W3b, b3b, gamma3, beta3,
           Wout, bout):
    we_s = jnp.stack([We1, We2, We3])
    be_s = jnp.stack([be1, be2, be3])
    e3 = _edge_mlp(edge_attr, we_s, be_s)
    zeros = jnp.zeros((N, D), jnp.float32)
    src = edge_index[0]
    dst = edge_index[1]

    layers = [
        (W1a, b1a, W1b, b1b, gamma1, beta1),
        (W2a, b2a, W2b, b2b, gamma2, beta2),
        (W3a, b3a, W3b, b3b, gamma3, beta3),
    ]
    h = x
    for l, (wa, ba, wb, bb, g, bt) in enumerate(layers):
        p = _EDGE_GS[l](h, e3, src, dst, zeros)
        y, st = _node_mlp(h, p, wa, ba, wb, bb)
        h = _norm(y, st, g, bt)

    return _pool(h, batch, Wout, bout)


# speedup vs baseline: 1.1155x; 1.1155x over previous
import functools

import jax
import jax.numpy as jnp
from jax import lax
from jax.experimental import pallas as pl
from jax.experimental.pallas import tpu as pltpu
from jax.experimental.pallas import tpu_sc as plsc

N = 10000
E = 320000
D = 128
H = 128
ED = 16
G = 64

NC = 2
NS = 16
NW = NC * NS
EPW = E // NW
C = 40
NCHUNK = EPW // C
RPS = 624
RTAIL = N - NS * RPS


def _make_edge_gather_scatter(layer: int):
    mesh = plsc.VectorSubcoreMesh(core_axis_name="c", subcore_axis_name="s")

    @functools.partial(
        pl.kernel,
        out_type=jax.ShapeDtypeStruct((NC, N, D), jnp.float32),
        mesh=mesh,
        scratch_types=[
            pltpu.VMEM((EPW,), jnp.int32),
            [pltpu.VMEM((C,), jnp.int32) for _ in range(3)],
            [pltpu.VMEM((C, D), jnp.float32) for _ in range(3)],
            [pltpu.VMEM((C, D), jnp.float32) for _ in range(3)],
            pltpu.VMEM_SHARED((N, D), jnp.float32),
            pltpu.SemaphoreType.DMA((5, 3)),
        ],
    )
    def k(h_hbm, e3_hbm, src_hbm, dst_hbm, z_hbm, out_hbm,
          srcall_v, dst_v, hrow_v, e_v, aggr_sh, sems):
        c = lax.axis_index("c")
        s = lax.axis_index("s")
        wid = s * NC + c
        base = wid * EPW
        pltpu.sync_copy(src_hbm.at[pl.ds(base, EPW)], srcall_v)
        pltpu.sync_copy(z_hbm.at[pl.ds(s * RPS, RPS)],
                        aggr_sh.at[pl.ds(s * RPS, RPS)])

        @pl.when(s == NS - 1)
        def _():
            pltpu.sync_copy(z_hbm.at[pl.ds(NS * RPS, RTAIL)],
                            aggr_sh.at[pl.ds(NS * RPS, RTAIL)])

        plsc.subcore_barrier()

        def gather_desc(j, b):
            return pltpu.make_async_copy(
                h_hbm.at[srcall_v.at[pl.ds(j * C, C)]], hrow_v[b],
                sems.at[0, b])

        def e_desc(j, b):
            return pltpu.make_async_copy(
                e3_hbm.at[layer, pl.ds(base + j * C, C)], e_v[b],
                sems.at[1, b])

        def dst_desc(j, b):
            return pltpu.make_async_copy(
                dst_hbm.at[pl.ds(base + j * C, C)], dst_v[b],
                sems.at[2, b])

        def scat_wait(b):
            pltpu.make_async_copy(
                hrow_v[b], aggr_sh.at[dst_v[b]], sems.at[3, b]).wait()

        def issue(j, b):
            dst_desc(j, b).start()
            e_desc(j, b).start()
            gather_desc(j, b).start()

        def consume(j, b):
            gather_desc(j, b).wait()
            e_desc(j, b).wait()

            @plsc.parallel_loop(0, C, unroll=4)
            def _(i):
                for col in range(D // 16):
                    sl = pl.ds(col * 16, 16)
                    hrow_v[b][i, sl] = jnp.maximum(
                        hrow_v[b][i, sl] + e_v[b][i, sl], 0.0)

            dst_desc(j, b).wait()
            pltpu.async_copy(hrow_v[b], aggr_sh.at[dst_v[b]],
                             sems.at[3, b], add=True)

        issue(0, 0)
        issue(1, 1)

        def trip(i, carry):
            j0 = 3 * i
            for t in range(3):
                j = j0 + t
                consume(j, t)
                b2 = (t + 2) % 3

                @pl.when((j >= 1) & (j + 2 < NCHUNK))
                def _():
                    scat_wait(b2)

                @pl.when(j + 2 < NCHUNK)
                def _():
                    issue(j + 2, b2)

            return carry

        lax.fori_loop(0, NCHUNK // 3, trip, 0)
        consume(NCHUNK - 1, (NCHUNK - 1) % 3)
        scat_wait((NCHUNK - 3) % 3)
        scat_wait((NCHUNK - 2) % 3)
        scat_wait((NCHUNK - 1) % 3)
        plsc.subcore_barrier()
        pltpu.sync_copy(aggr_sh.at[pl.ds(s * RPS, RPS)],
                        out_hbm.at[c, pl.ds(s * RPS, RPS)])

        @pl.when(s == NS - 1)
        def _():
            pltpu.sync_copy(aggr_sh.at[pl.ds(NS * RPS, RTAIL)],
                            out_hbm.at[c, pl.ds(NS * RPS, RTAIL)])

    return k


_EDGE_GS = [_make_edge_gather_scatter(l) for l in range(3)]


BE = 4000


def _edge_mlp_body(attr_ref, we_ref, be_ref, out_ref):
    a = attr_ref[...]
    for l in range(3):
        out_ref[l] = (jnp.dot(a, we_ref[l], preferred_element_type=jnp.float32)
                      + be_ref[l][None, :])


def _edge_mlp(edge_attr, we_s, be_s):
    return pl.pallas_call(
        _edge_mlp_body,
        grid=(E // BE,),
        in_specs=[
            pl.BlockSpec((BE, ED), lambda i: (i, 0)),
            pl.BlockSpec((3, ED, D), lambda i: (0, 0, 0)),
            pl.BlockSpec((3, D), lambda i: (0, 0)),
        ],
        out_specs=pl.BlockSpec((3, BE, D), lambda i: (0, i, 0)),
        out_shape=jax.ShapeDtypeStruct((3, E, D), jnp.float32),
    )(edge_attr, we_s, be_s)


BN = 1000


def _node_mlp_body(h_ref, p_ref, wa_ref, ba_ref, wb_ref, bb_ref,
                   y_ref, st_ref):
    i = pl.program_id(0)
    hpre = h_ref[...] + p_ref[0] + p_ref[1]
    t = jnp.maximum(
        jnp.dot(hpre, wa_ref[...], preferred_element_type=jnp.float32)
        + ba_ref[...], 0.0)
    y = jnp.dot(t, wb_ref[...], preferred_element_type=jnp.float32) + bb_ref[...]
    y_ref[...] = y
    s1 = jnp.sum(y, axis=0, keepdims=True)
    s2 = jnp.sum(y * y, axis=0, keepdims=True)
    upd = jnp.concatenate(
        [s1, s2, jnp.zeros((6, H), jnp.float32)], axis=0)

    @pl.when(i == 0)
    def _():
        st_ref[...] = upd

    @pl.when(i > 0)
    def _():
        st_ref[...] = st_ref[...] + upd


def _node_mlp(h, p, wa, ba, wb, bb):
    return pl.pallas_call(
        _node_mlp_body,
        grid=(N // BN,),
        in_specs=[
            pl.BlockSpec((BN, D), lambda i: (i, 0)),
            pl.BlockSpec((NC, BN, D), lambda i: (0, i, 0)),
            pl.BlockSpec((D, H), lambda i: (0, 0)),
            pl.BlockSpec((1, H), lambda i: (0, 0)),
            pl.BlockSpec((H, H), lambda i: (0, 0)),
            pl.BlockSpec((1, H), lambda i: (0, 0)),
        ],
        out_specs=[
            pl.BlockSpec((BN, H), lambda i: (i, 0)),
            pl.BlockSpec((8, H), lambda i: (0, 0)),
        ],
        out_shape=[
            jax.ShapeDtypeStruct((N, H), jnp.float32),
            jax.ShapeDtypeStruct((8, H), jnp.float32),
        ],
    )(h, p, wa, ba.reshape(1, H), wb, bb.reshape(1, H))


def _norm_body(y_ref, st_ref, g_ref, bt_ref, h_ref):
    mu = st_ref[0:1, :] * (1.0 / N)
    ex2 = st_ref[1:2, :] * (1.0 / N)
    var = ex2 - mu * mu
    scale = lax.rsqrt(var + 1e-5) * g_ref[...]
    h_ref[...] = jnp.maximum((y_ref[...] - mu) * scale + bt_ref[...], 0.0)


def _norm(y, st, g, bt):
    return pl.pallas_call(
        _norm_body,
        grid=(N // BN,),
        in_specs=[
            pl.BlockSpec((BN, H), lambda i: (i, 0)),
            pl.BlockSpec((8, H), lambda i: (0, 0)),
            pl.BlockSpec((1, H), lambda i: (0, 0)),
            pl.BlockSpec((1, H), lambda i: (0, 0)),
        ],
        out_specs=pl.BlockSpec((BN, H), lambda i: (i, 0)),
        out_shape=jax.ShapeDtypeStruct((N, H), jnp.float32),
    )(y, st, g.reshape(1, H), bt.reshape(1, H))


def _pool_body(h_ref, b_ref, wo_ref, bo_ref, out_ref, acc_ref, cnt_ref):
    i = pl.program_id(0)
    b = b_ref[0, 0, :]
    gids = lax.broadcasted_iota(jnp.int32, (G, BN), 0)
    onehot = (b[None, :] == gids).astype(jnp.float32)
    pacc = jnp.dot(onehot, h_ref[...], preferred_element_type=jnp.float32)
    pcnt = jnp.sum(onehot, axis=1, keepdims=True)

    @pl.when(i == 0)
    def _():
        acc_ref[...] = pacc
        cnt_ref[...] = pcnt

    @pl.when(i > 0)
    def _():
        acc_ref[...] = acc_ref[...] + pacc
        cnt_ref[...] = cnt_ref[...] + pcnt

    @pl.when(i == (N // BN) - 1)
    def _():
        pooled = acc_ref[...] / jnp.maximum(cnt_ref[...], 1.0)
        out_ref[...] = (jnp.dot(pooled, wo_ref[...],
                                preferred_element_type=jnp.float32)
                        + bo_ref[...])


def _pool(h, batch, wo, bo):
    b3 = batch.reshape(N // BN, 1, BN)
    return pl.pallas_call(
        _pool_body,
        grid=(N // BN,),
        in_specs=[
            pl.BlockSpec((BN, H), lambda i: (i, 0)),
            pl.BlockSpec((1, 1, BN), lambda i: (i, 0, 0)),
            pl.BlockSpec((H, 1), lambda i: (0, 0)),
            pl.BlockSpec((1, 1), lambda i: (0, 0)),
        ],
        out_specs=pl.BlockSpec((G, 1), lambda i: (0, 0)),
        out_shape=jax.ShapeDtypeStruct((G, 1), jnp.float32),
        scratch_shapes=[
            pltpu.VMEM((G, H), jnp.float32),
            pltpu.VMEM((G, 1), jnp.float32),
        ],
    )(h, b3, wo, bo.reshape(1, 1))


def kernel(x, edge_index, edge_attr, batch,
           We1, be1, W1a, b1a, W1b, b1b, gamma1, beta1,
           We2, be2, W2a, b2a, W2b, b2b, gamma2, beta2,
           We3, be3, W3a, b3a, W3b, b3b, gamma3, beta3,
           Wout, bout):
    we_s = jnp.stack([We1, We2, We3])
    be_s = jnp.stack([be1, be2, be3])
    e3 = _edge_mlp(edge_attr, we_s, be_s)
    zeros = jnp.zeros((N, D), jnp.float32)
    src = edge_index[0]
    dst = edge_index[1]

    layers = [
        (W1a, b1a, W1b, b1b, gamma1, beta1),
        (W2a, b2a, W2b, b2b, gamma2, beta2),
        (W3a, b3a, W3b, b3b, gamma3, beta3),
    ]
    h = x
    for l, (wa, ba, wb, bb, g, bt) in enumerate(layers):
        p = _EDGE_GS[l](h, e3, src, dst, zeros)
        y, st = _node_mlp(h, p, wa, ba, wb, bb)
        h = _norm(y, st, g, bt)

    return _pool(h, batch, Wout, bout)
